# baseline (speedup 1.0000x reference)
import jax
import jax.numpy as jnp
from jax.experimental import pallas as pl
from jax.experimental.pallas import tpu as pltpu

_DIM = 4096
_BM = 1024
_BN = 512


def _mm_kernel(j_ref, e_ref, o_ref):
    o_ref[...] = jnp.dot(j_ref[...], e_ref[...],
                         preferred_element_type=jnp.float32)


def kernel(eye, jacobian):
    J = jacobian.reshape(_DIM, _DIM)
    E = eye.reshape(_DIM, _DIM)
    out = pl.pallas_call(
        _mm_kernel,
        grid=(_DIM // _BM, _DIM // _BN),
        in_specs=[
            pl.BlockSpec((_BM, _DIM), lambda i, j: (i, 0)),
            pl.BlockSpec((_DIM, _BN), lambda i, j: (0, j)),
        ],
        out_specs=pl.BlockSpec((_BM, _BN), lambda i, j: (i, j)),
        out_shape=jax.ShapeDtypeStruct((_DIM, _DIM), jnp.float32),
        compiler_params=pltpu.CompilerParams(
            dimension_semantics=("parallel", "arbitrary"),
            vmem_limit_bytes=56 * 1024 * 1024,
        ),
    )(J, E)
    return out.reshape(_DIM * _DIM)
